# baseline (speedup 1.0000x reference)
import functools

import jax
import jax.numpy as jnp
from jax import lax
from jax.experimental import pallas as pl
from jax.experimental.pallas import tpu as pltpu
from jax.experimental.pallas import tpu_sc as plsc

BT = 128
FT = 1024
BTT = 256


def _gate_kernel(x_ref, wg_ref, i0_ref, i1_ref, w0_ref, w1_ref):
    x = x_ref[...]
    wg = wg_ref[...]
    logits = jax.lax.dot_general(
        x, wg, (((1,), (1,)), ((), ())), preferred_element_type=jnp.float32
    )
    T, E = logits.shape
    iota = jax.lax.broadcasted_iota(jnp.int32, (T, E), 1)
    m0 = jnp.max(logits, axis=1, keepdims=True)
    i0 = jnp.min(jnp.where(logits == m0, iota, E), axis=1, keepdims=True)
    masked = jnp.where(iota == i0, -jnp.inf, logits)
    m1 = jnp.max(masked, axis=1, keepdims=True)
    i1 = jnp.min(jnp.where(masked == m1, iota, E), axis=1, keepdims=True)
    z = jnp.exp(m1 - m0)
    w0 = 1.0 / (1.0 + z)
    i0_ref[...] = i0
    i1_ref[...] = i1
    w0_ref[...] = w0
    w1_ref[...] = 1.0 - w0


def _route_sc_kernel(i0_hbm, i1_hbm, w0_hbm, w1_hbm, tok_hbm, wp_hbm,
                     offp_hbm, i0_v, i1_v, w0_v, w1_v, tok_v, wp_v, cnt_v,
                     out16_v, *, T, TKP, E):
    @pl.when((lax.axis_index("c") == 0) & (lax.axis_index("s") == 0))
    def _():
        pltpu.sync_copy(i0_hbm, i0_v)
        pltpu.sync_copy(i1_hbm, i1_v)
        pltpu.sync_copy(w0_hbm, w0_v)
        pltpu.sync_copy(w1_hbm, w1_v)
        zeros16 = jnp.zeros((16,), jnp.int32)
        cnt_v[...] = zeros16

        def zbody(i, c):
            tok_v[pl.ds(i * 16, 16)] = zeros16
            wp_v[pl.ds(i * 16, 16)] = jnp.zeros((16,), jnp.float32)
            return c
        lax.fori_loop(0, TKP // 16, zbody, 0)

        def count1(ev):
            rank, lastm = plsc.scan_count(ev)
            plsc.addupdate_scatter(cnt_v, [ev], rank, mask=lastm)

        def cbody(i, c):
            count1(i0_v[pl.ds(i * 16, 16)])
            count1(i1_v[pl.ds(i * 16, 16)])
            return c
        lax.fori_loop(0, T // 16, cbody, 0)

        cnt = cnt_v[...]
        padded = ((cnt + (BT - 1)) // BT) * BT
        incl = plsc.cumsum(padded)
        excl = incl - padded
        out16_v[...] = excl
        pltpu.sync_copy(out16_v, offp_hbm)
        cnt_v[...] = excl

        iota16 = lax.broadcasted_iota(jnp.int32, (16,), 0)

        def place(ev, wv, tokv):
            rank, lastm = plsc.scan_count(ev)
            base = plsc.load_gather(cnt_v, [ev])
            pos = jnp.minimum(base + rank - 1, TKP - 1)
            plsc.store_scatter(tok_v, [pos], tokv)
            plsc.store_scatter(wp_v, [pos], wv)
            plsc.addupdate_scatter(cnt_v, [ev], rank, mask=lastm)

        def pbody(i, c):
            tokv = iota16 + i * 16
            place(i0_v[pl.ds(i * 16, 16)], w0_v[pl.ds(i * 16, 16)], tokv)
            place(i1_v[pl.ds(i * 16, 16)], w1_v[pl.ds(i * 16, 16)], tokv)
            return c
        lax.fori_loop(0, T // 16, pbody, 0)

        pltpu.sync_copy(tok_v, tok_hbm)
        pltpu.sync_copy(wp_v, wp_hbm)


def _route_sc(i0, i1, w0, w1v, T, TKP, E):
    mesh = plsc.VectorSubcoreMesh(core_axis_name="c", subcore_axis_name="s")
    fn = functools.partial(
        pl.kernel,
        out_type=[
            jax.ShapeDtypeStruct((TKP,), jnp.int32),
            jax.ShapeDtypeStruct((TKP,), jnp.float32),
            jax.ShapeDtypeStruct((16,), jnp.int32),
        ],
        mesh=mesh,
        compiler_params=pltpu.CompilerParams(needs_layout_passes=False),
        scratch_types=[
            pltpu.VMEM((T,), jnp.int32),
            pltpu.VMEM((T,), jnp.int32),
            pltpu.VMEM((T,), jnp.float32),
            pltpu.VMEM((T,), jnp.float32),
            pltpu.VMEM((TKP,), jnp.int32),
            pltpu.VMEM((TKP,), jnp.float32),
            pltpu.VMEM((16,), jnp.int32),
            pltpu.VMEM((16,), jnp.int32),
        ],
    )(functools.partial(_route_sc_kernel, T=T, TKP=TKP, E=E))
    return fn(i0.reshape(T), i1.reshape(T), w0.reshape(T), w1v.reshape(T))


def _moe_ffn_kernel(offp_s, tok_ref, wv_ref, xb_ref, w1_ref, b1_ref, w2_ref,
                    b2_ref, yw_ref, xe_ref, y_ref, *, nf, T):
    e = pl.program_id(0)
    f = pl.program_id(1)
    TKP = yw_ref.shape[0]
    s_raw = jnp.clip(offp_s[e], 0, TKP - BT)
    start = pl.multiple_of((s_raw // BT) * BT, BT)
    nch = jnp.minimum(
        jnp.clip((offp_s[e + 1] - offp_s[e]) // BT, 0, T // BT),
        (TKP - start) // BT)

    @pl.when((e == 0) & (f == 0))
    def _():
        yw_ref[...] = jnp.zeros_like(yw_ref)

    @pl.when(f == 0)
    def _():
        xb = xb_ref[...]

        def gbody(c, carry):
            p = pl.multiple_of(start + c * BT, BT)
            tok = tok_ref[pl.ds(p, BT), :]
            iota = jax.lax.broadcasted_iota(jnp.int32, (BT, T), 1)
            oh = (iota == tok).astype(jnp.bfloat16)
            xs = jnp.dot(oh, xb, preferred_element_type=jnp.float32)
            xe_ref[pl.ds(c * BT, BT), :] = xs.astype(jnp.bfloat16)
            return carry
        jax.lax.fori_loop(0, nch, gbody, 0)

    w1 = w1_ref[0]
    w2 = w2_ref[0]
    b1 = b1_ref[0, 0]

    def cbody(c, carry):
        xs = xe_ref[pl.ds(c * BT, BT), :]
        h = jnp.dot(xs, w1, preferred_element_type=jnp.float32) + b1
        h = 0.5 * h * (1.0 + jax.lax.erf(h * 0.7071067811865476))
        yp = jnp.dot(h.astype(jnp.bfloat16), w2,
                     preferred_element_type=jnp.float32)
        prev = jnp.where(f == 0, 0.0, y_ref[pl.ds(c * BT, BT), :])
        y_ref[pl.ds(c * BT, BT), :] = prev + yp
        return carry
    jax.lax.fori_loop(0, nch, cbody, 0)

    @pl.when(f == nf - 1)
    def _():
        b2 = b2_ref[0]

        def sbody(c, carry):
            p = pl.multiple_of(start + c * BT, BT)
            w = wv_ref[pl.ds(p, BT), :]
            row = (y_ref[pl.ds(c * BT, BT), :] + b2) * w
            yw_ref[pl.ds(p, BT), :] = row.astype(jnp.bfloat16)
            return carry
        jax.lax.fori_loop(0, nch, sbody, 0)


def _combine_kernel(tokr_ref, yw_ref, out_ref, *, TKP):
    i = pl.program_id(0)
    tok = tokr_ref[...]
    iota = jax.lax.broadcasted_iota(jnp.int32, (BTT, TKP), 0) + i * BTT
    oh2 = (iota == tok).astype(jnp.bfloat16)
    out_ref[...] = jnp.dot(oh2, yw_ref[...],
                           preferred_element_type=jnp.float32)


def kernel(x, Wg, W1, b1, W2, b2):
    B, S, D = x.shape
    E, _, F = W1.shape
    T = B * S
    K = 2
    TK = T * K
    nf = F // FT
    NBLK = TK // BT + E
    TKP = NBLK * BT
    x_flat = x.reshape(T, D)

    i0, i1, w0, w1v = pl.pallas_call(
        _gate_kernel,
        out_shape=[
            jax.ShapeDtypeStruct((T, 1), jnp.int32),
            jax.ShapeDtypeStruct((T, 1), jnp.int32),
            jax.ShapeDtypeStruct((T, 1), jnp.float32),
            jax.ShapeDtypeStruct((T, 1), jnp.float32),
        ],
    )(x_flat, Wg)

    tok_pad, w_pad, offp16 = _route_sc(i0, i1, w0, w1v, T, TKP, E)
    offp = offp16[: E + 1]

    xb = x_flat.astype(jnp.bfloat16)
    yw = pl.pallas_call(
        functools.partial(_moe_ffn_kernel, nf=nf, T=T),
        grid_spec=pltpu.PrefetchScalarGridSpec(
            num_scalar_prefetch=1,
            grid=(E, nf),
            in_specs=[
                pl.BlockSpec((TKP, 1), lambda e, f, *_: (0, 0)),
                pl.BlockSpec((TKP, 1), lambda e, f, *_: (0, 0)),
                pl.BlockSpec((T, D), lambda e, f, *_: (0, 0)),
                pl.BlockSpec((1, D, FT), lambda e, f, *_: (e, 0, f)),
                pl.BlockSpec((1, 1, 1, FT), lambda e, f, *_: (e, f, 0, 0)),
                pl.BlockSpec((1, FT, D), lambda e, f, *_: (e, f, 0)),
                pl.BlockSpec((1, 1, D), lambda e, f, *_: (e, 0, 0)),
            ],
            out_specs=pl.BlockSpec((TKP, D), lambda e, f, *_: (0, 0)),
            scratch_shapes=[
                pltpu.VMEM((T, D), jnp.bfloat16),
                pltpu.VMEM((T, D), jnp.float32),
            ],
        ),
        out_shape=jax.ShapeDtypeStruct((TKP, D), jnp.bfloat16),
        compiler_params=pltpu.CompilerParams(
            dimension_semantics=("arbitrary", "arbitrary"),
        ),
    )(offp, tok_pad.reshape(TKP, 1), w_pad.reshape(TKP, 1), xb,
      W1.astype(jnp.bfloat16), b1.reshape(E, nf, 1, FT),
      W2.astype(jnp.bfloat16), b2.reshape(E, 1, D))

    out = pl.pallas_call(
        functools.partial(_combine_kernel, TKP=TKP),
        grid=(T // BTT,),
        in_specs=[
            pl.BlockSpec((1, TKP), lambda i: (0, 0)),
            pl.BlockSpec((TKP, D), lambda i: (0, 0)),
        ],
        out_specs=pl.BlockSpec((BTT, D), lambda i: (i, 0)),
        out_shape=jax.ShapeDtypeStruct((T, D), jnp.float32),
    )(tok_pad.reshape(1, TKP), yw)

    return out.reshape(B, S, D)


# speedup vs baseline: 1.0691x; 1.0691x over previous
import functools

import jax
import jax.numpy as jnp
from jax import lax
from jax.experimental import pallas as pl
from jax.experimental.pallas import tpu as pltpu
from jax.experimental.pallas import tpu_sc as plsc

BT = 256
FT = 1024


def _gate_kernel(x_ref, wg_ref, i0_ref, i1_ref, w0_ref, w1_ref):
    x = x_ref[...]
    wg = wg_ref[...]
    logits = jax.lax.dot_general(
        x, wg, (((1,), (1,)), ((), ())), preferred_element_type=jnp.float32
    )
    T, E = logits.shape
    iota = jax.lax.broadcasted_iota(jnp.int32, (T, E), 1)
    m0 = jnp.max(logits, axis=1, keepdims=True)
    i0 = jnp.min(jnp.where(logits == m0, iota, E), axis=1, keepdims=True)
    masked = jnp.where(iota == i0, -jnp.inf, logits)
    m1 = jnp.max(masked, axis=1, keepdims=True)
    i1 = jnp.min(jnp.where(masked == m1, iota, E), axis=1, keepdims=True)
    z = jnp.exp(m1 - m0)
    w0 = 1.0 / (1.0 + z)
    i0_ref[...] = i0
    i1_ref[...] = i1
    w0_ref[...] = w0
    w1_ref[...] = 1.0 - w0


def _route_sc_kernel(i0_hbm, i1_hbm, w0_hbm, w1_hbm, tok_hbm, wp_hbm,
                     offp_hbm, i0_v, i1_v, w0_v, w1_v, tok_v, wp_v, cnt_v,
                     out16_v, *, T, TKP, E):
    @pl.when((lax.axis_index("c") == 0) & (lax.axis_index("s") == 0))
    def _():
        pltpu.sync_copy(i0_hbm, i0_v)
        pltpu.sync_copy(i1_hbm, i1_v)
        pltpu.sync_copy(w0_hbm, w0_v)
        pltpu.sync_copy(w1_hbm, w1_v)
        zeros16 = jnp.zeros((16,), jnp.int32)
        cnt_v[...] = zeros16

        def zbody(i, c):
            tok_v[pl.ds(i * 16, 16)] = zeros16
            wp_v[pl.ds(i * 16, 16)] = jnp.zeros((16,), jnp.float32)
            return c
        lax.fori_loop(0, TKP // 16, zbody, 0)

        def count1(ev):
            rank, lastm = plsc.scan_count(ev)
            plsc.addupdate_scatter(cnt_v, [ev], rank, mask=lastm)

        def cbody(i, c):
            count1(i0_v[pl.ds(i * 16, 16)])
            count1(i1_v[pl.ds(i * 16, 16)])
            return c
        lax.fori_loop(0, T // 16, cbody, 0)

        cnt = cnt_v[...]
        padded = ((cnt + (BT - 1)) // BT) * BT
        incl = plsc.cumsum(padded)
        excl = incl - padded
        out16_v[...] = excl
        pltpu.sync_copy(out16_v, offp_hbm)
        cnt_v[...] = excl

        iota16 = lax.broadcasted_iota(jnp.int32, (16,), 0)

        def place(ev, wv, tokv):
            rank, lastm = plsc.scan_count(ev)
            base = plsc.load_gather(cnt_v, [ev])
            pos = jnp.minimum(base + rank - 1, TKP - 1)
            plsc.store_scatter(tok_v, [pos], tokv)
            plsc.store_scatter(wp_v, [pos], wv)
            plsc.addupdate_scatter(cnt_v, [ev], rank, mask=lastm)

        def pbody(i, c):
            tokv = iota16 + i * 16
            place(i0_v[pl.ds(i * 16, 16)], w0_v[pl.ds(i * 16, 16)], tokv)
            place(i1_v[pl.ds(i * 16, 16)], w1_v[pl.ds(i * 16, 16)], tokv)
            return c
        lax.fori_loop(0, T // 16, pbody, 0)

        pltpu.sync_copy(tok_v, tok_hbm)
        pltpu.sync_copy(wp_v, wp_hbm)


def _route_sc(i0, i1, w0, w1v, T, TKP, E):
    mesh = plsc.VectorSubcoreMesh(core_axis_name="c", subcore_axis_name="s")
    fn = functools.partial(
        pl.kernel,
        out_type=[
            jax.ShapeDtypeStruct((TKP,), jnp.int32),
            jax.ShapeDtypeStruct((TKP,), jnp.float32),
            jax.ShapeDtypeStruct((16,), jnp.int32),
        ],
        mesh=mesh,
        compiler_params=pltpu.CompilerParams(needs_layout_passes=False),
        scratch_types=[
            pltpu.VMEM((T,), jnp.int32),
            pltpu.VMEM((T,), jnp.int32),
            pltpu.VMEM((T,), jnp.float32),
            pltpu.VMEM((T,), jnp.float32),
            pltpu.VMEM((TKP,), jnp.int32),
            pltpu.VMEM((TKP,), jnp.float32),
            pltpu.VMEM((16,), jnp.int32),
            pltpu.VMEM((16,), jnp.int32),
        ],
    )(functools.partial(_route_sc_kernel, T=T, TKP=TKP, E=E))
    return fn(i0.reshape(T), i1.reshape(T), w0.reshape(T), w1v.reshape(T))


def _moe_ffn_kernel(offp_s, tok_ref, wv_ref, xb_ref, w1_ref, b1_ref, w2_ref,
                    b2_ref, out_ref, xe_ref, y_ref, *, nf, T, TKP):
    e = pl.program_id(0)
    f = pl.program_id(1)
    s_raw = jnp.clip(offp_s[e], 0, TKP - BT)
    start = pl.multiple_of((s_raw // BT) * BT, BT)
    nch = jnp.minimum(
        jnp.clip((offp_s[e + 1] - offp_s[e]) // BT, 0, T // BT),
        (TKP - start) // BT)

    @pl.when((e == 0) & (f == 0))
    def _():
        out_ref[...] = jnp.zeros_like(out_ref)

    @pl.when(f == 0)
    def _():
        xb = xb_ref[...]

        def gbody(c, carry):
            p = pl.multiple_of(start + c * BT, BT)
            tok = tok_ref[pl.ds(p, BT), :]
            iota = jax.lax.broadcasted_iota(jnp.int32, (BT, T), 1)
            oh = (iota == tok).astype(jnp.bfloat16)
            xs = jnp.dot(oh, xb, preferred_element_type=jnp.float32)
            xe_ref[pl.ds(c * BT, BT), :] = xs.astype(jnp.bfloat16)
            return carry
        jax.lax.fori_loop(0, nch, gbody, 0)

    w1 = w1_ref[0]
    w2 = w2_ref[0]
    b1 = b1_ref[0, 0]

    def cbody(c, carry):
        xs = xe_ref[pl.ds(c * BT, BT), :]
        h = jnp.dot(xs, w1, preferred_element_type=jnp.float32) + b1
        h = 0.5 * h * (1.0 + jax.lax.erf(h * 0.7071067811865476))
        yp = jnp.dot(h.astype(jnp.bfloat16), w2,
                     preferred_element_type=jnp.float32)
        prev = jnp.where(f == 0, 0.0, y_ref[pl.ds(c * BT, BT), :])
        y_ref[pl.ds(c * BT, BT), :] = prev + yp
        return carry
    jax.lax.fori_loop(0, nch, cbody, 0)

    @pl.when(f == nf - 1)
    def _():
        b2 = b2_ref[0]

        def sbody(c, carry):
            p = pl.multiple_of(start + c * BT, BT)
            w = wv_ref[pl.ds(p, BT), :]
            tok = tok_ref[pl.ds(p, BT), :]
            row = ((y_ref[pl.ds(c * BT, BT), :] + b2) * w).astype(jnp.bfloat16)
            iota = jax.lax.broadcasted_iota(jnp.int32, (T, BT), 0)
            oh = (iota == tok.reshape(1, BT)).astype(jnp.bfloat16)
            out_ref[...] += jnp.dot(oh, row,
                                    preferred_element_type=jnp.float32)
            return carry
        jax.lax.fori_loop(0, nch, sbody, 0)


def kernel(x, Wg, W1, b1, W2, b2):
    B, S, D = x.shape
    E, _, F = W1.shape
    T = B * S
    K = 2
    TK = T * K
    nf = F // FT
    NBLK = TK // BT + E
    TKP = NBLK * BT
    x_flat = x.reshape(T, D)

    i0, i1, w0, w1v = pl.pallas_call(
        _gate_kernel,
        out_shape=[
            jax.ShapeDtypeStruct((T, 1), jnp.int32),
            jax.ShapeDtypeStruct((T, 1), jnp.int32),
            jax.ShapeDtypeStruct((T, 1), jnp.float32),
            jax.ShapeDtypeStruct((T, 1), jnp.float32),
        ],
    )(x_flat, Wg)

    tok_pad, w_pad, offp16 = _route_sc(i0, i1, w0, w1v, T, TKP, E)
    offp = offp16[: E + 1]

    xb = x_flat.astype(jnp.bfloat16)
    out = pl.pallas_call(
        functools.partial(_moe_ffn_kernel, nf=nf, T=T, TKP=TKP),
        grid_spec=pltpu.PrefetchScalarGridSpec(
            num_scalar_prefetch=1,
            grid=(E, nf),
            in_specs=[
                pl.BlockSpec((TKP, 1), lambda e, f, *_: (0, 0)),
                pl.BlockSpec((TKP, 1), lambda e, f, *_: (0, 0)),
                pl.BlockSpec((T, D), lambda e, f, *_: (0, 0)),
                pl.BlockSpec((1, D, FT), lambda e, f, *_: (e, 0, f)),
                pl.BlockSpec((1, 1, 1, FT), lambda e, f, *_: (e, f, 0, 0)),
                pl.BlockSpec((1, FT, D), lambda e, f, *_: (e, f, 0)),
                pl.BlockSpec((1, 1, D), lambda e, f, *_: (e, 0, 0)),
            ],
            out_specs=pl.BlockSpec((T, D), lambda e, f, *_: (0, 0)),
            scratch_shapes=[
                pltpu.VMEM((T, D), jnp.bfloat16),
                pltpu.VMEM((T, D), jnp.float32),
            ],
        ),
        out_shape=jax.ShapeDtypeStruct((T, D), jnp.float32),
        compiler_params=pltpu.CompilerParams(
            dimension_semantics=("arbitrary", "arbitrary"),
        ),
    )(offp, tok_pad.reshape(TKP, 1), w_pad.reshape(TKP, 1), xb,
      W1.astype(jnp.bfloat16), b1.reshape(E, nf, 1, FT),
      W2.astype(jnp.bfloat16), b2.reshape(E, 1, D))

    return out.reshape(B, S, D)
